# baseline (speedup 1.0000x reference)
import functools

import jax
import jax.numpy as jnp
from jax import lax
from jax.experimental import pallas as pl
from jax.experimental.pallas import tpu as pltpu
from jax.experimental.pallas import tpu_sc as plsc

D = 64
B = 4096
_STRIPE = 128
_NSB = 4

_info = plsc.get_sparse_core_info()
_NC = _info.num_cores
_NS = _info.num_subcores
_NW = _NC * _NS
_BPW = B // _NW

_mesh = plsc.VectorSubcoreMesh(core_axis_name="c", subcore_axis_name="s")


def _stage_idx(idx_hbm, iv, base):
    pltpu.sync_copy(idx_hbm.at[pl.ds(base, _BPW)], iv)


def _fire_rows(tab, iv, rv, sem):
    def body(g, carry):
        v = iv[pl.ds(g * 16, 16)]
        for j in range(16):
            row = v[j]
            pltpu.async_copy(tab.at[row], rv.at[g * 16 + j], sem)
        return carry

    lax.fori_loop(0, _BPW // 16, body, 0)


def _drain_rows(out_slice, rv, sem):
    pltpu.make_async_copy(out_slice, rv, sem).wait()
    pltpu.sync_copy(rv, out_slice)


@functools.partial(
    pl.kernel,
    mesh=_mesh,
    compiler_params=pltpu.CompilerParams(needs_layout_passes=False),
    out_type=jax.ShapeDtypeStruct((3, B, D), jnp.float32),
    scratch_types=(
        [pltpu.VMEM((_BPW,), jnp.int32) for _ in range(3)]
        + [pltpu.VMEM((_BPW, D), jnp.float32) for _ in range(3)]
        + [pltpu.VMEM((D, _STRIPE), jnp.float32) for _ in range(_NSB)]
        + [pltpu.SemaphoreType.DMA for _ in range(3)]
        + [pltpu.SemaphoreType.DMA for _ in range(_NSB)]
    ),
)
def _gather_a(i_cs, i_o, i_bt, t_cs, t_ot, t_bt, out_hbm,
              x0, x1, x2, r0, r1, r2,
              sb0, sb1, sb2, sb3,
              s0, s1, s2, q0, q1, q2, q3):
    wid = lax.axis_index("s") * _NC + lax.axis_index("c")
    base = wid * _BPW
    sbufs = (sb0, sb1, sb2, sb3)
    qsems = (q0, q1, q2, q3)
    _stage_idx(i_cs, x0, base)
    _stage_idx(i_o, x1, base)
    _stage_idx(i_bt, x2, base)
    _fire_rows(t_cs, x0, r0, s0)
    _fire_rows(t_bt, x2, r2, s2)

    jvecs = [lax.iota(jnp.int32, 16) + 16 * k for k in range(4)]

    def _extract(lane, buf, i):
        lvec = jnp.full((16,), lane, dtype=jnp.int32)
        for k in range(4):
            col = plsc.load_gather(buf, [jvecs[k], lvec])
            r1[i, pl.ds(k * 16, 16)] = col

    def _stripe_body(g, carry):
        v = x1[pl.ds(g * 16, 16)]
        pend = []
        for j in range(16):
            row = v[j]
            base_lane = pl.multiple_of((row // _STRIPE) * _STRIPE, _STRIPE)
            lane = row - base_lane
            nb = j % _NSB
            if j >= _NSB:
                plane, pcopy = pend[j - _NSB]
                pcopy.wait()
                _extract(plane, sbufs[nb], g * 16 + (j - _NSB))
            cp = pltpu.async_copy(
                t_ot.at[:, pl.ds(base_lane, _STRIPE)], sbufs[nb], qsems[nb])
            pend.append((lane, cp))
        for j in range(16 - _NSB, 16):
            plane, pcopy = pend[j]
            pcopy.wait()
            _extract(plane, sbufs[j % _NSB], g * 16 + j)
        return carry

    lax.fori_loop(0, _BPW // 16, _stripe_body, 0)
    pltpu.sync_copy(r1, out_hbm.at[1, pl.ds(base, _BPW)])

    _drain_rows(out_hbm.at[0, pl.ds(base, _BPW)], r0, s0)
    _drain_rows(out_hbm.at[2, pl.ds(base, _BPW)], r2, s2)


@functools.partial(
    pl.kernel,
    mesh=_mesh,
    compiler_params=pltpu.CompilerParams(needs_layout_passes=False),
    out_type=jax.ShapeDtypeStruct((2, B, D), jnp.float32),
    scratch_types=(
        [pltpu.VMEM((_BPW,), jnp.int32) for _ in range(2)]
        + [pltpu.VMEM((_BPW, D), jnp.float32) for _ in range(2)]
        + [pltpu.SemaphoreType.DMA for _ in range(2)]
    ),
)
def _gather_b(i_c, i_dp, t_c, t_dp, out_hbm, x0, x1, r0, r1, s0, s1):
    wid = lax.axis_index("s") * _NC + lax.axis_index("c")
    base = wid * _BPW
    _stage_idx(i_c, x0, base)
    _stage_idx(i_dp, x1, base)
    _fire_rows(t_c, x0, r0, s0)
    _fire_rows(t_dp, x1, r1, s1)
    _drain_rows(out_hbm.at[0, pl.ds(base, _BPW)], r0, s0)
    _drain_rows(out_hbm.at[1, pl.ds(base, _BPW)], r1, s1)


_BLK = 512


def _proj_body(emb_a_ref, emb_b_ref, wt_ref, b_ref, out_ref):
    ea = emb_a_ref[...]
    eb = emb_b_ref[...]
    x_cs, x_o, x_bt = ea[0], ea[1], ea[2]
    x_c, x_dp = eb[0], eb[1]
    wt = wt_ref[...]
    bias = b_ref[...]
    y_c = jnp.dot(x_c, wt[0], preferred_element_type=jnp.float32) + bias[0] + x_cs
    y_o = jnp.dot(x_o, wt[1], preferred_element_type=jnp.float32) + bias[1] + x_dp
    y_dp = jnp.dot(x_dp, wt[2], preferred_element_type=jnp.float32) + bias[2] + x_bt
    out_ref[...] = jnp.concatenate([y_c, x_cs, y_o, y_dp, x_bt], axis=-1)


_proj = pl.pallas_call(
    _proj_body,
    grid=(B // _BLK,),
    in_specs=[
        pl.BlockSpec((3, _BLK, D), lambda i: (0, i, 0)),
        pl.BlockSpec((2, _BLK, D), lambda i: (0, i, 0)),
        pl.BlockSpec((3, D, D), lambda i: (0, 0, 0)),
        pl.BlockSpec((3, D), lambda i: (0, 0)),
    ],
    out_specs=pl.BlockSpec((_BLK, 5 * D), lambda i: (i, 0)),
    out_shape=jax.ShapeDtypeStruct((B, 5 * D), jnp.float32),
)


def kernel(campaignid, campaignsetid, offerid, demand_pkgname, business_type,
           table_campaignid, table_campaignsetid, table_offerid,
           table_demand_pkgname, table_business_type,
           W_campaignid, b_campaignid, W_offerid, b_offerid,
           W_demand_pkgname, b_demand_pkgname):
    i_c = campaignid.astype(jnp.int32)
    i_cs = campaignsetid.astype(jnp.int32)
    i_o = offerid.astype(jnp.int32)
    i_dp = demand_pkgname.astype(jnp.int32)
    i_bt = business_type.astype(jnp.int32)
    emb_a = _gather_a(i_cs, i_o, i_bt,
                      table_campaignsetid, table_offerid.T, table_business_type)
    emb_b = _gather_b(i_c, i_dp, table_campaignid, table_demand_pkgname)
    wt = jnp.stack([W_campaignid.T, W_offerid.T, W_demand_pkgname.T])
    bias = jnp.stack([b_campaignid, b_offerid, b_demand_pkgname])
    return _proj(emb_a, emb_b, wt, bias)
